# Initial kernel scaffold; baseline (speedup 1.0000x reference)
import functools

import jax
import jax.numpy as jnp
from jax import lax
from jax.experimental import pallas as pl
from jax.experimental.pallas import tpu as pltpu
from jax.experimental.pallas import tpu_sc as plsc

N_CORES = 2
N_SUBCORES = 16
LANES = 16
NW = N_CORES * N_SUBCORES


@functools.lru_cache(maxsize=None)
def _build(n_nodes: int, n_edges: int, d_feat: int):
    E, D = n_edges, d_feat
    assert E % NW == 0 and D % LANES == 0
    EPW = E // NW
    C = 100
    assert EPW % C == 0 and C % 4 == 0
    NCHUNK = EPW // C
    NGRP = (C + LANES - 1) // LANES
    CP = NGRP * LANES
    UNROLL_A = 4
    assert C % UNROLL_A == 0
    NJ = D // LANES

    mesh = plsc.VectorSubcoreMesh(core_axis_name="c", subcore_axis_name="s")

    @functools.partial(
        pl.kernel,
        mesh=mesh,
        out_type=jax.ShapeDtypeStruct((E,), jnp.float32),
        scratch_types=[
            pltpu.VMEM((C,), jnp.int32),
            pltpu.VMEM((C,), jnp.int32),
            pltpu.VMEM((C, D), jnp.float32),
            pltpu.VMEM((C, D), jnp.float32),
            pltpu.VMEM((CP, LANES), jnp.float32),
            pltpu.VMEM((EPW + LANES,), jnp.float32),
            pltpu.SemaphoreType.DMA,
            pltpu.SemaphoreType.DMA,
        ],
    )
    def sc_kernel(h_hbm, src_hbm, dst_hbm, out_hbm,
                  sidx, didx, srows, drows, part, outv, sem_a, sem_b):
        wid = lax.axis_index("s") * N_CORES + lax.axis_index("c")
        ebase = wid * EPW
        iota = lax.iota(jnp.int32, LANES)

        def do_chunk(g, carry):
            base = ebase + g * C
            pltpu.sync_copy(src_hbm.at[pl.ds(base, C)], sidx)
            pltpu.sync_copy(dst_hbm.at[pl.ds(base, C)], didx)
            cp_s = pltpu.async_copy(h_hbm.at[sidx], srows, sem_a)
            cp_d = pltpu.async_copy(h_hbm.at[didx], drows, sem_b)
            cp_s.wait()
            cp_d.wait()

            def pass_a(i0, c):
                for u in range(UNROLL_A):
                    i = i0 * UNROLL_A + u
                    acc = srows[i, pl.ds(0, LANES)] * drows[i, pl.ds(0, LANES)]
                    for j in range(1, NJ):
                        acc = acc + (srows[i, pl.ds(j * LANES, LANES)]
                                     * drows[i, pl.ds(j * LANES, LANES)])
                    part[i, :] = acc
                return c

            lax.fori_loop(0, C // UNROLL_A, pass_a, 0)

            def pass_b(q, c):
                rows = q * LANES + iota
                tot = plsc.load_gather(part, [rows, jnp.zeros((LANES,), jnp.int32)])
                for cc in range(1, LANES):
                    tot = tot + plsc.load_gather(
                        part, [rows, jnp.full((LANES,), cc, jnp.int32)])
                outv[pl.ds(g * C + q * LANES, LANES)] = tot
                return c

            lax.fori_loop(0, NGRP, pass_b, 0)
            return carry

        lax.fori_loop(0, NCHUNK, do_chunk, 0)
        pltpu.sync_copy(outv.at[pl.ds(0, EPW)], out_hbm.at[pl.ds(ebase, EPW)])

    return sc_kernel


def kernel(h, edge_index):
    src = edge_index[0].astype(jnp.int32)
    dst = edge_index[1].astype(jnp.int32)
    score = _build(h.shape[0], edge_index.shape[1], h.shape[1])(h, src, dst)
    return score.reshape(-1, 1)


# speedup vs baseline: 3.4866x; 3.4866x over previous
import functools

import jax
import jax.numpy as jnp
from jax import lax
from jax.experimental import pallas as pl
from jax.experimental.pallas import tpu as pltpu
from jax.experimental.pallas import tpu_sc as plsc

N_CORES = 2
N_SUBCORES = 16
LANES = 16
NW = N_CORES * N_SUBCORES


@functools.lru_cache(maxsize=None)
def _build(n_nodes: int, n_edges: int, d_feat: int):
    E, D = n_edges, d_feat
    assert E % NW == 0 and D % LANES == 0
    EPW = E // NW
    C = 80
    assert EPW % C == 0 and C % 8 == 0
    NCHUNK = EPW // C
    NGRP = (C + LANES - 1) // LANES
    CP = NGRP * LANES
    UNROLL_A = 4
    assert C % UNROLL_A == 0
    NJ = D // LANES

    mesh = plsc.VectorSubcoreMesh(core_axis_name="c", subcore_axis_name="s")

    @functools.partial(
        pl.kernel,
        mesh=mesh,
        out_type=jax.ShapeDtypeStruct((E,), jnp.float32),
        compiler_params=pltpu.CompilerParams(needs_layout_passes=False),
        scratch_types=[
            pltpu.VMEM((C,), jnp.int32),
            pltpu.VMEM((C,), jnp.int32),
            pltpu.VMEM((C, D), jnp.float32),
            pltpu.VMEM((C, D), jnp.float32),
            pltpu.VMEM((CP * LANES,), jnp.float32),
            pltpu.VMEM((EPW + LANES,), jnp.float32),
            pltpu.SemaphoreType.DMA,
            pltpu.SemaphoreType.DMA,
        ],
    )
    def sc_kernel(h_hbm, src_hbm, dst_hbm, out_hbm,
                  sidx, didx, srows, drows, part, outv, sem_a, sem_b):
        wid = lax.axis_index("s") * N_CORES + lax.axis_index("c")
        ebase = wid * EPW
        iota = lax.iota(jnp.int32, LANES)

        def do_chunk(g, carry):
            base = ebase + g * C
            pltpu.sync_copy(src_hbm.at[pl.ds(base, C)], sidx)
            pltpu.sync_copy(dst_hbm.at[pl.ds(base, C)], didx)
            cp_s = pltpu.async_copy(h_hbm.at[sidx], srows, sem_a)
            cp_d = pltpu.async_copy(h_hbm.at[didx], drows, sem_b)
            cp_s.wait()
            cp_d.wait()

            def pass_a(i0, c):
                for u in range(UNROLL_A):
                    i = i0 * UNROLL_A + u
                    acc = srows[i, pl.ds(0, LANES)] * drows[i, pl.ds(0, LANES)]
                    for j in range(1, NJ):
                        acc = acc + (srows[i, pl.ds(j * LANES, LANES)]
                                     * drows[i, pl.ds(j * LANES, LANES)])
                    part[pl.ds(i * LANES, LANES)] = acc
                return c

            lax.fori_loop(0, C // UNROLL_A, pass_a, 0)

            def pass_b(q, c):
                flat = (q * LANES + iota) * LANES
                tot = plsc.load_gather(part, [flat])
                for cc in range(1, LANES):
                    tot = tot + plsc.load_gather(part, [flat + cc])
                outv[pl.ds(g * C + q * LANES, LANES)] = tot
                return c

            lax.fori_loop(0, NGRP, pass_b, 0)
            return carry

        lax.fori_loop(0, NCHUNK, do_chunk, 0)
        pltpu.sync_copy(outv.at[pl.ds(0, EPW)], out_hbm.at[pl.ds(ebase, EPW)])

    return sc_kernel


def kernel(h, edge_index):
    src = edge_index[0].astype(jnp.int32)
    dst = edge_index[1].astype(jnp.int32)
    score = _build(h.shape[0], edge_index.shape[1], h.shape[1])(h, src, dst)
    return score.reshape(-1, 1)
